# baseline (speedup 1.0000x reference)
import functools

import jax
import jax.numpy as jnp
from jax import lax
from jax.experimental import pallas as pl
from jax.experimental.pallas import tpu as pltpu
from jax.experimental.pallas import tpu_sc as plsc

B, N, D = 1, 2048, 1024
H = 2048
E = 8
T = B * N

BT = 128
NB = (2 * T + 1792) // BT
CAP = T
XS = E * CAP + BT
GARBAGE = E * CAP // BT

CH = 512
NCH = T // CH

_INV_SQRT2 = 0.7071067811865476


def _router_body(x_ref, wr_ref,
                 d1_ref, d2_ref, w1r_ref, w2r_ref,
                 se_ref, rs_ref, re_ref, aux_ref, carry, prob):
    c = pl.program_id(0)

    @pl.when(c == 0)
    def _zero():
        carry[...] = jnp.zeros_like(carry)
        prob[...] = jnp.zeros_like(prob)

    logits = jax.lax.dot_general(
        x_ref[...], wr_ref[...], (((1,), (0,)), ((), ())),
        preferred_element_type=jnp.float32)
    m = jnp.max(logits, axis=1, keepdims=True)
    p = jnp.exp(logits - m)
    gates = p / jnp.sum(p, axis=1, keepdims=True)

    tio = jax.lax.broadcasted_iota(jnp.int32, (CH, E), 1)
    c1 = jnp.max(gates, axis=1, keepdims=True)
    j1 = jnp.min(jnp.where(gates == c1, tio, E), axis=1, keepdims=True)
    g2 = jnp.where(tio == j1, -jnp.inf, gates)
    c2 = jnp.max(g2, axis=1, keepdims=True)
    j2 = jnp.min(jnp.where(g2 == c2, tio, E), axis=1, keepdims=True)
    dd = jnp.maximum(c1 + c2, 1e-9)

    sel = jnp.where(tio == j1, 1.0, 0.0) + jnp.where(tio == j2, 1.0, 0.0)
    ltri = jnp.where(
        jax.lax.broadcasted_iota(jnp.int32, (CH, CH), 0)
        > jax.lax.broadcasted_iota(jnp.int32, (CH, CH), 1), 1.0, 0.0)
    ranks = jax.lax.dot_general(
        ltri, sel, (((1,), (0,)), ((), ())),
        preferred_element_type=jnp.float32) + carry[...]

    r1 = jnp.sum(jnp.where(tio == j1, ranks, 0.0), axis=1, keepdims=True)
    r2 = jnp.sum(jnp.where(tio == j2, ranks, 0.0), axis=1, keepdims=True)
    d1_ref[...] = (j1 * CAP + r1.astype(jnp.int32))
    d2_ref[...] = (j2 * CAP + r2.astype(jnp.int32))
    ones128 = jnp.ones((1, 128), jnp.float32)
    w1r_ref[...] = (c1 / dd) * ones128
    w2r_ref[...] = (c2 / dd) * ones128

    carry[...] += jnp.sum(sel, axis=0, keepdims=True)
    prob[...] += jnp.sum(gates, axis=0, keepdims=True)

    @pl.when(c == NCH - 1)
    def _fin():
        counts = carry[...]
        aux_ref[0, 0] = E * jnp.sum((prob[...] / T) * (counts / T))
        eio = jax.lax.broadcasted_iota(jnp.int32, (1, E), 1)
        cs = [jnp.sum(jnp.where(eio == k, counts, 0.0), keepdims=True)
              for k in range(E)]
        nb = [jnp.floor((ck + (BT - 1)) / BT) for ck in cs]
        bs = [jnp.zeros((1, 1), jnp.float32)]
        for k in range(1, E):
            bs.append(bs[k - 1] + nb[k - 1])

        bio = (jax.lax.broadcasted_iota(jnp.int32, (1, NB), 1)
               .astype(jnp.float32))
        se = jnp.zeros((1, NB), jnp.float32)
        for k in range(E):
            se = se + jnp.where(bio >= bs[k], 1.0, 0.0)
        se = se - 1.0
        rs = jnp.zeros((1, NB), jnp.float32)
        re = jnp.zeros((1, NB), jnp.float32)
        for k in range(E):
            rs = rs + jnp.where(se == k, k * CAP + (bio - bs[k]) * BT, 0.0)
            re = re + jnp.where(se == k, k * CAP + cs[k], 0.0)
        rs = jnp.where(rs < re, rs, float(E * CAP))
        se_ref[...] = se.astype(jnp.int32)
        rs_ref[...] = rs.astype(jnp.int32)
        re_ref[...] = re.astype(jnp.int32)


def _router(xf, wr):
    col_i = jax.ShapeDtypeStruct((T, 1), jnp.int32)
    col_f = jax.ShapeDtypeStruct((T, 1), jnp.float32)
    return pl.pallas_call(
        _router_body,
        grid=(NCH,),
        in_specs=[
            pl.BlockSpec((CH, D), lambda c: (c, 0)),
            pl.BlockSpec((D, E), lambda c: (0, 0)),
        ],
        out_specs=(
            pl.BlockSpec((CH, 1), lambda c: (c, 0)),
            pl.BlockSpec((CH, 1), lambda c: (c, 0)),
            pl.BlockSpec((CH, 128), lambda c: (c, 0)),
            pl.BlockSpec((CH, 128), lambda c: (c, 0)),
            pl.BlockSpec((1, NB), lambda c: (0, 0)),
            pl.BlockSpec((1, NB), lambda c: (0, 0)),
            pl.BlockSpec((1, NB), lambda c: (0, 0)),
            pl.BlockSpec(memory_space=pltpu.SMEM),
        ),
        out_shape=(col_i, col_i,
                   jax.ShapeDtypeStruct((T, 128), jnp.float32),
                   jax.ShapeDtypeStruct((T, 128), jnp.float32),
                   jax.ShapeDtypeStruct((1, NB), jnp.int32),
                   jax.ShapeDtypeStruct((1, NB), jnp.int32),
                   jax.ShapeDtypeStruct((1, NB), jnp.int32),
                   jax.ShapeDtypeStruct((1, 1), jnp.float32)),
        scratch_shapes=[pltpu.VMEM((1, E), jnp.float32),
                        pltpu.VMEM((1, E), jnp.float32)],
    )(xf, wr)


NC = 2
NS = 16
NW = NC * NS
TPW = T // NW


def _dispatch(xf, d1, d2, w1r, w2r):
    mesh = plsc.VectorSubcoreMesh(core_axis_name="c", subcore_axis_name="s")

    @functools.partial(
        pl.kernel, mesh=mesh,
        out_type=(jax.ShapeDtypeStruct((XS, D), jnp.float32),
                  jax.ShapeDtypeStruct((XS, 128), jnp.float32)),
        scratch_types=[pltpu.VMEM((TPW,), jnp.int32),
                       pltpu.VMEM((TPW,), jnp.int32),
                       pltpu.VMEM((TPW, D), jnp.float32),
                       pltpu.VMEM((TPW, 128), jnp.float32)],
    )
    def k(x_hbm, d1_hbm, d2_hbm, w1r_hbm, w2r_hbm, xs_hbm, ws_hbm,
          idx1_v, idx2_v, rows_v, wbuf_v):
        wid = lax.axis_index("s") * NC + lax.axis_index("c")
        base = wid * TPW
        pltpu.sync_copy(d1_hbm.at[wid], idx1_v)
        pltpu.sync_copy(d2_hbm.at[wid], idx2_v)
        pltpu.sync_copy(x_hbm.at[pl.ds(base, TPW)], rows_v)
        pltpu.sync_copy(rows_v, xs_hbm.at[idx1_v])
        pltpu.sync_copy(rows_v, xs_hbm.at[idx2_v])
        pltpu.sync_copy(w1r_hbm.at[pl.ds(base, TPW)], wbuf_v)
        pltpu.sync_copy(wbuf_v, ws_hbm.at[idx1_v])
        pltpu.sync_copy(w2r_hbm.at[pl.ds(base, TPW)], wbuf_v)
        pltpu.sync_copy(wbuf_v, ws_hbm.at[idx2_v])

    return k(xf, d1.reshape(NW, TPW), d2.reshape(NW, TPW), w1r, w2r)


def _gmm_body(se_ref, rs_ref, re_ref, xs_ref, w1_ref, b1_ref, w2_ref,
              b2_ref, ws_ref, out_ref):
    b = pl.program_id(0)
    rstart = rs_ref[b]
    rend = re_ref[b]

    @pl.when(rstart < rend)
    def _compute():
        rows = rstart + jax.lax.broadcasted_iota(jnp.int32, (BT, 1), 0)
        xm = jnp.where(rows < rend, xs_ref[...], 0.0)
        h = jax.lax.dot_general(
            xm, w1_ref[0], (((1,), (0,)), ((), ())),
            preferred_element_type=jnp.float32) + b1_ref[0]
        h = 0.5 * h * (1.0 + jax.lax.erf(h * _INV_SQRT2))
        o = jax.lax.dot_general(
            h, w2_ref[0], (((1,), (0,)), ((), ())),
            preferred_element_type=jnp.float32) + b2_ref[0]
        out_ref[...] = o * ws_ref[:, 0:1]


def _gmm(xs, ws, w1, b1, w2, b2, se, rs, re):
    grid_spec = pltpu.PrefetchScalarGridSpec(
        num_scalar_prefetch=3,
        grid=(NB,),
        in_specs=[
            pl.BlockSpec((BT, D), lambda b, se, rs, re: (rs[b] // BT, 0)),
            pl.BlockSpec((1, D, H), lambda b, se, rs, re: (se[b], 0, 0)),
            pl.BlockSpec((1, 1, H), lambda b, se, rs, re: (se[b], 0, 0)),
            pl.BlockSpec((1, H, D), lambda b, se, rs, re: (se[b], 0, 0)),
            pl.BlockSpec((1, 1, D), lambda b, se, rs, re: (se[b], 0, 0)),
            pl.BlockSpec((BT, 128), lambda b, se, rs, re: (rs[b] // BT, 0)),
        ],
        out_specs=pl.BlockSpec((BT, D), lambda b, se, rs, re: (rs[b] // BT, 0)),
    )
    return pl.pallas_call(
        _gmm_body,
        grid_spec=grid_spec,
        out_shape=jax.ShapeDtypeStruct((XS, D), jnp.float32),
    )(se, rs, re, xs, w1, b1.reshape(E, 1, H), w2, b2.reshape(E, 1, D), ws)


CPW = 16
NCHK = TPW // CPW


def _combine(o, d1, d2):
    mesh = plsc.VectorSubcoreMesh(core_axis_name="c", subcore_axis_name="s")

    @functools.partial(
        pl.kernel, mesh=mesh,
        out_type=jax.ShapeDtypeStruct((T, D), jnp.float32),
        scratch_types=[pltpu.VMEM((TPW,), jnp.int32),
                       pltpu.VMEM((TPW,), jnp.int32),
                       pltpu.VMEM((CPW, D), jnp.float32),
                       pltpu.VMEM((CPW, D), jnp.float32),
                       pltpu.VMEM((CPW, D), jnp.float32),
                       pltpu.VMEM((CPW, D), jnp.float32),
                       pltpu.SemaphoreType.DMA,
                       pltpu.SemaphoreType.DMA,
                       pltpu.SemaphoreType.DMA],
    )
    def k(o_hbm, d1_hbm, d2_hbm, out_hbm, idx1_v, idx2_v,
          r1a, r2a, r1b, r2b, sem_a, sem_b, sem_out):
        wid = lax.axis_index("s") * NC + lax.axis_index("c")
        base = wid * TPW
        pltpu.sync_copy(d1_hbm.at[pl.ds(base, TPW)], idx1_v)
        pltpu.sync_copy(d2_hbm.at[pl.ds(base, TPW)], idx2_v)
        bufs = [(r1a, r2a, sem_a), (r1b, r2b, sem_b)]

        def issue(c):
            r1, r2, sem = bufs[c % 2]
            cp1 = pltpu.async_copy(
                o_hbm.at[idx1_v.at[pl.ds(c * CPW, CPW)]], r1, sem)
            cp2 = pltpu.async_copy(
                o_hbm.at[idx2_v.at[pl.ds(c * CPW, CPW)]], r2, sem)
            return cp1, cp2

        pend = issue(0)
        stores = [None] * NCHK
        for c in range(NCHK):
            cp1, cp2 = pend
            cp1.wait()
            cp2.wait()
            if c + 1 < NCHK:
                if c >= 1:
                    stores[c - 1].wait()
                pend = issue(c + 1)
            r1, r2, _ = bufs[c % 2]

            def body(i, _):
                def inner(kk, _):
                    for u in range(8):
                        sl = pl.ds((kk * 8 + u) * 16, 16)
                        r1[i, sl] = r1[i, sl] + r2[i, sl]
                    return 0
                return lax.fori_loop(0, D // 128, inner, 0)

            lax.fori_loop(0, CPW, body, 0)
            stores[c] = pltpu.async_copy(
                r1, out_hbm.at[pl.ds(base + c * CPW, CPW)], sem_out)
        stores[NCHK - 2].wait()
        stores[NCHK - 1].wait()

    return k(o, d1.reshape(T), d2.reshape(T))


def kernel(x, Wr, W1, b1, W2, b2):
    xf = x.reshape(T, D)
    d1, d2, w1r, w2r, se, rs, re, aux = _router(xf, Wr)
    xs, ws = _dispatch(xf, d1, d2, w1r, w2r)
    o = _gmm(xs, ws, W1, b1, W2, b2,
             se.reshape(NB), rs.reshape(NB), re.reshape(NB))
    out = _combine(o, d1, d2)
    return out.reshape(B, N, D), aux[0, 0]


# speedup vs baseline: 1.0532x; 1.0532x over previous
import functools

import jax
import jax.numpy as jnp
from jax import lax
from jax.experimental import pallas as pl
from jax.experimental.pallas import tpu as pltpu
from jax.experimental.pallas import tpu_sc as plsc

B, N, D = 1, 2048, 1024
H = 2048
E = 8
T = B * N

BT = 256
NB = (2 * T + 1792) // BT
CAP = T
XS = E * CAP + BT
GARBAGE = E * CAP // BT

CH = 512
NCH = T // CH

_INV_SQRT2 = 0.7071067811865476


def _router_body(x_ref, wr_ref,
                 d1_ref, d2_ref, w1r_ref, w2r_ref,
                 se_ref, rs_ref, re_ref, aux_ref, carry, prob):
    c = pl.program_id(0)

    @pl.when(c == 0)
    def _zero():
        carry[...] = jnp.zeros_like(carry)
        prob[...] = jnp.zeros_like(prob)

    logits = jax.lax.dot_general(
        x_ref[...], wr_ref[...], (((1,), (0,)), ((), ())),
        preferred_element_type=jnp.float32)
    m = jnp.max(logits, axis=1, keepdims=True)
    p = jnp.exp(logits - m)
    gates = p / jnp.sum(p, axis=1, keepdims=True)

    tio = jax.lax.broadcasted_iota(jnp.int32, (CH, E), 1)
    c1 = jnp.max(gates, axis=1, keepdims=True)
    j1 = jnp.min(jnp.where(gates == c1, tio, E), axis=1, keepdims=True)
    g2 = jnp.where(tio == j1, -jnp.inf, gates)
    c2 = jnp.max(g2, axis=1, keepdims=True)
    j2 = jnp.min(jnp.where(g2 == c2, tio, E), axis=1, keepdims=True)
    dd = jnp.maximum(c1 + c2, 1e-9)

    sel = jnp.where(tio == j1, 1.0, 0.0) + jnp.where(tio == j2, 1.0, 0.0)
    ltri = jnp.where(
        jax.lax.broadcasted_iota(jnp.int32, (CH, CH), 0)
        > jax.lax.broadcasted_iota(jnp.int32, (CH, CH), 1), 1.0, 0.0)
    ranks = jax.lax.dot_general(
        ltri, sel, (((1,), (0,)), ((), ())),
        preferred_element_type=jnp.float32) + carry[...]

    r1 = jnp.sum(jnp.where(tio == j1, ranks, 0.0), axis=1, keepdims=True)
    r2 = jnp.sum(jnp.where(tio == j2, ranks, 0.0), axis=1, keepdims=True)
    d1_ref[...] = (j1 * CAP + r1.astype(jnp.int32))
    d2_ref[...] = (j2 * CAP + r2.astype(jnp.int32))
    ones128 = jnp.ones((1, 128), jnp.float32)
    w1r_ref[...] = (c1 / dd) * ones128
    w2r_ref[...] = (c2 / dd) * ones128

    carry[...] += jnp.sum(sel, axis=0, keepdims=True)
    prob[...] += jnp.sum(gates, axis=0, keepdims=True)

    @pl.when(c == NCH - 1)
    def _fin():
        counts = carry[...]
        aux_ref[0, 0] = E * jnp.sum((prob[...] / T) * (counts / T))
        eio = jax.lax.broadcasted_iota(jnp.int32, (1, E), 1)
        cs = [jnp.sum(jnp.where(eio == k, counts, 0.0), keepdims=True)
              for k in range(E)]
        nb = [jnp.floor((ck + (BT - 1)) / BT) for ck in cs]
        bs = [jnp.zeros((1, 1), jnp.float32)]
        for k in range(1, E):
            bs.append(bs[k - 1] + nb[k - 1])

        bio = (jax.lax.broadcasted_iota(jnp.int32, (1, NB), 1)
               .astype(jnp.float32))
        se = jnp.zeros((1, NB), jnp.float32)
        for k in range(E):
            se = se + jnp.where(bio >= bs[k], 1.0, 0.0)
        se = se - 1.0
        rs = jnp.zeros((1, NB), jnp.float32)
        re = jnp.zeros((1, NB), jnp.float32)
        for k in range(E):
            rs = rs + jnp.where(se == k, k * CAP + (bio - bs[k]) * BT, 0.0)
            re = re + jnp.where(se == k, k * CAP + cs[k], 0.0)
        rs = jnp.where(rs < re, rs, float(E * CAP))
        se_ref[...] = se.astype(jnp.int32)
        rs_ref[...] = rs.astype(jnp.int32)
        re_ref[...] = re.astype(jnp.int32)


def _router(xf, wr):
    col_i = jax.ShapeDtypeStruct((T, 1), jnp.int32)
    col_f = jax.ShapeDtypeStruct((T, 1), jnp.float32)
    return pl.pallas_call(
        _router_body,
        grid=(NCH,),
        in_specs=[
            pl.BlockSpec((CH, D), lambda c: (c, 0)),
            pl.BlockSpec((D, E), lambda c: (0, 0)),
        ],
        out_specs=(
            pl.BlockSpec((CH, 1), lambda c: (c, 0)),
            pl.BlockSpec((CH, 1), lambda c: (c, 0)),
            pl.BlockSpec((CH, 128), lambda c: (c, 0)),
            pl.BlockSpec((CH, 128), lambda c: (c, 0)),
            pl.BlockSpec((1, NB), lambda c: (0, 0)),
            pl.BlockSpec((1, NB), lambda c: (0, 0)),
            pl.BlockSpec((1, NB), lambda c: (0, 0)),
            pl.BlockSpec(memory_space=pltpu.SMEM),
        ),
        out_shape=(col_i, col_i,
                   jax.ShapeDtypeStruct((T, 128), jnp.float32),
                   jax.ShapeDtypeStruct((T, 128), jnp.float32),
                   jax.ShapeDtypeStruct((1, NB), jnp.int32),
                   jax.ShapeDtypeStruct((1, NB), jnp.int32),
                   jax.ShapeDtypeStruct((1, NB), jnp.int32),
                   jax.ShapeDtypeStruct((1, 1), jnp.float32)),
        scratch_shapes=[pltpu.VMEM((1, E), jnp.float32),
                        pltpu.VMEM((1, E), jnp.float32)],
    )(xf, wr)


NC = 2
NS = 16
NW = NC * NS
TPW = T // NW


def _dispatch(xf, d1, d2, w1r, w2r):
    mesh = plsc.VectorSubcoreMesh(core_axis_name="c", subcore_axis_name="s")

    @functools.partial(
        pl.kernel, mesh=mesh,
        out_type=(jax.ShapeDtypeStruct((XS, D), jnp.float32),
                  jax.ShapeDtypeStruct((XS, 128), jnp.float32)),
        scratch_types=[pltpu.VMEM((TPW,), jnp.int32),
                       pltpu.VMEM((TPW,), jnp.int32),
                       pltpu.VMEM((TPW, D), jnp.float32),
                       pltpu.VMEM((TPW, 128), jnp.float32)],
    )
    def k(x_hbm, d1_hbm, d2_hbm, w1r_hbm, w2r_hbm, xs_hbm, ws_hbm,
          idx1_v, idx2_v, rows_v, wbuf_v):
        wid = lax.axis_index("s") * NC + lax.axis_index("c")
        base = wid * TPW
        pltpu.sync_copy(d1_hbm.at[wid], idx1_v)
        pltpu.sync_copy(d2_hbm.at[wid], idx2_v)
        pltpu.sync_copy(x_hbm.at[pl.ds(base, TPW)], rows_v)
        pltpu.sync_copy(rows_v, xs_hbm.at[idx1_v])
        pltpu.sync_copy(rows_v, xs_hbm.at[idx2_v])
        pltpu.sync_copy(w1r_hbm.at[pl.ds(base, TPW)], wbuf_v)
        pltpu.sync_copy(wbuf_v, ws_hbm.at[idx1_v])
        pltpu.sync_copy(w2r_hbm.at[pl.ds(base, TPW)], wbuf_v)
        pltpu.sync_copy(wbuf_v, ws_hbm.at[idx2_v])

    return k(xf, d1.reshape(NW, TPW), d2.reshape(NW, TPW), w1r, w2r)


def _gmm_body(se_ref, rs_ref, re_ref, xs_ref, w1_ref, b1_ref, w2_ref,
              b2_ref, ws_ref, out_ref):
    b = pl.program_id(0)
    rstart = rs_ref[b]
    rend = re_ref[b]

    @pl.when(rstart < rend)
    def _compute():
        rows = rstart + jax.lax.broadcasted_iota(jnp.int32, (BT, 1), 0)
        xm = jnp.where(rows < rend, xs_ref[...], 0.0)
        h = jax.lax.dot_general(
            xm, w1_ref[0], (((1,), (0,)), ((), ())),
            preferred_element_type=jnp.float32) + b1_ref[0]
        h = 0.5 * h * (1.0 + jax.lax.erf(h * _INV_SQRT2))
        o = jax.lax.dot_general(
            h, w2_ref[0], (((1,), (0,)), ((), ())),
            preferred_element_type=jnp.float32) + b2_ref[0]
        out_ref[...] = o * ws_ref[:, 0:1]


def _gmm(xs, ws, w1, b1, w2, b2, se, rs, re):
    grid_spec = pltpu.PrefetchScalarGridSpec(
        num_scalar_prefetch=3,
        grid=(NB,),
        in_specs=[
            pl.BlockSpec((BT, D), lambda b, se, rs, re: (rs[b] // BT, 0)),
            pl.BlockSpec((1, D, H), lambda b, se, rs, re: (se[b], 0, 0)),
            pl.BlockSpec((1, 1, H), lambda b, se, rs, re: (se[b], 0, 0)),
            pl.BlockSpec((1, H, D), lambda b, se, rs, re: (se[b], 0, 0)),
            pl.BlockSpec((1, 1, D), lambda b, se, rs, re: (se[b], 0, 0)),
            pl.BlockSpec((BT, 128), lambda b, se, rs, re: (rs[b] // BT, 0)),
        ],
        out_specs=pl.BlockSpec((BT, D), lambda b, se, rs, re: (rs[b] // BT, 0)),
    )
    return pl.pallas_call(
        _gmm_body,
        grid_spec=grid_spec,
        out_shape=jax.ShapeDtypeStruct((XS, D), jnp.float32),
    )(se, rs, re, xs, w1, b1.reshape(E, 1, H), w2, b2.reshape(E, 1, D), ws)


CPW = 16
NCHK = TPW // CPW


def _combine(o, d1, d2):
    mesh = plsc.VectorSubcoreMesh(core_axis_name="c", subcore_axis_name="s")

    @functools.partial(
        pl.kernel, mesh=mesh,
        out_type=jax.ShapeDtypeStruct((T, D), jnp.float32),
        scratch_types=[pltpu.VMEM((TPW,), jnp.int32),
                       pltpu.VMEM((TPW,), jnp.int32),
                       pltpu.VMEM((CPW, D), jnp.float32),
                       pltpu.VMEM((CPW, D), jnp.float32),
                       pltpu.VMEM((CPW, D), jnp.float32),
                       pltpu.VMEM((CPW, D), jnp.float32),
                       pltpu.SemaphoreType.DMA,
                       pltpu.SemaphoreType.DMA,
                       pltpu.SemaphoreType.DMA],
    )
    def k(o_hbm, d1_hbm, d2_hbm, out_hbm, idx1_v, idx2_v,
          r1a, r2a, r1b, r2b, sem_a, sem_b, sem_out):
        wid = lax.axis_index("s") * NC + lax.axis_index("c")
        base = wid * TPW
        pltpu.sync_copy(d1_hbm.at[pl.ds(base, TPW)], idx1_v)
        pltpu.sync_copy(d2_hbm.at[pl.ds(base, TPW)], idx2_v)
        bufs = [(r1a, r2a, sem_a), (r1b, r2b, sem_b)]

        def issue(c):
            r1, r2, sem = bufs[c % 2]
            cp1 = pltpu.async_copy(
                o_hbm.at[idx1_v.at[pl.ds(c * CPW, CPW)]], r1, sem)
            cp2 = pltpu.async_copy(
                o_hbm.at[idx2_v.at[pl.ds(c * CPW, CPW)]], r2, sem)
            return cp1, cp2

        pend = issue(0)
        stores = [None] * NCHK
        for c in range(NCHK):
            cp1, cp2 = pend
            cp1.wait()
            cp2.wait()
            if c + 1 < NCHK:
                if c >= 1:
                    stores[c - 1].wait()
                pend = issue(c + 1)
            r1, r2, _ = bufs[c % 2]

            def body(i, _):
                def inner(kk, _):
                    for u in range(8):
                        sl = pl.ds((kk * 8 + u) * 16, 16)
                        r1[i, sl] = r1[i, sl] + r2[i, sl]
                    return 0
                return lax.fori_loop(0, D // 128, inner, 0)

            lax.fori_loop(0, CPW, body, 0)
            stores[c] = pltpu.async_copy(
                r1, out_hbm.at[pl.ds(base + c * CPW, CPW)], sem_out)
        stores[NCHK - 2].wait()
        stores[NCHK - 1].wait()

    return k(o, d1.reshape(T), d2.reshape(T))


def kernel(x, Wr, W1, b1, W2, b2):
    xf = x.reshape(T, D)
    d1, d2, w1r, w2r, se, rs, re, aux = _router(xf, Wr)
    xs, ws = _dispatch(xf, d1, d2, w1r, w2r)
    o = _gmm(xs, ws, W1, b1, W2, b2,
             se.reshape(NB), rs.reshape(NB), re.reshape(NB))
    out = _combine(o, d1, d2)
    return out.reshape(B, N, D), aux[0, 0]
